# baseline (speedup 1.0000x reference)
import functools

import jax
import jax.numpy as jnp
from jax import lax
from jax.experimental import pallas as pl
from jax.experimental.pallas import tpu as pltpu
from jax.experimental.pallas import tpu_sc as plsc

NC = 2
NS = 16
NW = NC * NS
L = 16

BATCH = 16384
K = 32
BPW = BATCH // NW
CHUNK = 128
NCHUNK = BPW // CHUNK


def _gmf_body(uid_hbm, iid_hbm, p_hbm, q_hbm, out_hbm,
              uidx_v, iidx_v, pu_v, qi_v, o_v, sem):
    wid = lax.axis_index("s") * NC + lax.axis_index("c")
    base = wid * BPW

    for j in range(NCHUNK):
        pltpu.sync_copy(uid_hbm.at[pl.ds(base + j * CHUNK, CHUNK)],
                        uidx_v.at[j])
        pltpu.sync_copy(iid_hbm.at[pl.ds(base + j * CHUNK, CHUNK)],
                        iidx_v.at[j])

    copies = []
    for j in range(NCHUNK):
        copies.append(pltpu.async_copy(
            p_hbm.at[uidx_v.at[j]], pu_v.at[pl.ds(j * CHUNK, CHUNK)], sem))
        copies.append(pltpu.async_copy(
            q_hbm.at[iidx_v.at[j]], qi_v.at[pl.ds(j * CHUNK, CHUNK)], sem))
    for c in copies:
        c.wait()

    lane = lax.iota(jnp.int32, L)

    def group(g, carry):
        row = g * L + lane

        def col_step(t, acc):
            col = (lane + t) & (K - 1)
            pv = plsc.load_gather(pu_v, [row, col])
            qv = plsc.load_gather(qi_v, [row, col])
            return acc + pv * qv

        acc = lax.fori_loop(0, K, col_step, jnp.zeros((L,), jnp.float32))
        o_v[pl.ds(g * L, L)] = acc
        return carry

    lax.fori_loop(0, BPW // L, group, 0)

    pltpu.sync_copy(o_v, out_hbm.at[pl.ds(base, BPW)])


@jax.jit
def _gmf(user_ids, item_ids, P, Q):
    mesh = plsc.VectorSubcoreMesh(
        core_axis_name="c", subcore_axis_name="s",
        num_cores=NC, num_subcores=NS)
    run = pl.kernel(
        _gmf_body,
        out_type=jax.ShapeDtypeStruct((BATCH,), jnp.float32),
        mesh=mesh,
        compiler_params=pltpu.CompilerParams(
            needs_layout_passes=False, use_tc_tiling_on_sc=False),
        scratch_types=[
            pltpu.VMEM((NCHUNK, CHUNK), jnp.int32),
            pltpu.VMEM((NCHUNK, CHUNK), jnp.int32),
            pltpu.VMEM((BPW, K), jnp.float32),
            pltpu.VMEM((BPW, K), jnp.float32),
            pltpu.VMEM((BPW,), jnp.float32),
            pltpu.SemaphoreType.DMA,
        ],
    )
    return run(user_ids, item_ids, P, Q)


def kernel(user_ids, item_ids, P, Q):
    out = _gmf(user_ids, item_ids, P, Q)
    return out.reshape(BATCH, 1)


# speedup vs baseline: 1.5039x; 1.5039x over previous
import functools

import jax
import jax.numpy as jnp
from jax import lax
from jax.experimental import pallas as pl
from jax.experimental.pallas import tpu as pltpu
from jax.experimental.pallas import tpu_sc as plsc

NC = 2
NS = 16
NW = NC * NS
L = 16

BATCH = 16384
K = 32
BPW = BATCH // NW
HALF = BPW // 2


def _gmf_body(uid_hbm, iid_hbm, p_hbm, q_hbm, out_hbm,
              uid_v, iid_v, pu_v, qi_v, o_v, sem):
    wid = lax.axis_index("s") * NC + lax.axis_index("c")
    base = wid * BPW

    pltpu.sync_copy(uid_hbm.at[pl.ds(base, BPW)], uid_v)
    pltpu.sync_copy(iid_hbm.at[pl.ds(base, BPW)], iid_v)

    lane = lax.iota(jnp.int32, L)

    def half(h, carry):
        hbase = h * HALF

        def fire(g, c):
            uvec = uid_v[pl.ds(hbase + g * L, L)]
            ivec = iid_v[pl.ds(hbase + g * L, L)]
            for j in range(L):
                b = g * L + j
                pltpu.async_copy(p_hbm.at[uvec[j]], pu_v.at[b], sem)
                pltpu.async_copy(q_hbm.at[ivec[j]], qi_v.at[b], sem)
            return c

        lax.fori_loop(0, HALF // L, fire, 0)

        def drain(b, c):
            pltpu.make_async_copy(p_hbm.at[0], pu_v.at[b], sem).wait()
            pltpu.make_async_copy(q_hbm.at[0], qi_v.at[b], sem).wait()
            return c

        lax.fori_loop(0, HALF, drain, 0)

        def group(g, c):
            row = g * L + lane

            def col_step(t, acc):
                col = (lane + t) & (K - 1)
                pv = plsc.load_gather(pu_v, [row, col])
                qv = plsc.load_gather(qi_v, [row, col])
                return acc + pv * qv

            acc = lax.fori_loop(0, K, col_step, jnp.zeros((L,), jnp.float32))
            o_v[pl.ds(hbase + g * L, L)] = acc
            return c

        lax.fori_loop(0, HALF // L, group, 0)
        return carry

    lax.fori_loop(0, BPW // HALF, half, 0)

    pltpu.sync_copy(o_v, out_hbm.at[pl.ds(base, BPW)])


@jax.jit
def _gmf(user_ids, item_ids, P, Q):
    mesh = plsc.VectorSubcoreMesh(
        core_axis_name="c", subcore_axis_name="s",
        num_cores=NC, num_subcores=NS)
    run = pl.kernel(
        _gmf_body,
        out_type=jax.ShapeDtypeStruct((BATCH,), jnp.float32),
        mesh=mesh,
        compiler_params=pltpu.CompilerParams(needs_layout_passes=False),
        scratch_types=[
            pltpu.VMEM((BPW,), jnp.int32),
            pltpu.VMEM((BPW,), jnp.int32),
            pltpu.VMEM((HALF, K), jnp.float32),
            pltpu.VMEM((HALF, K), jnp.float32),
            pltpu.VMEM((BPW,), jnp.float32),
            pltpu.SemaphoreType.DMA,
        ],
    )
    return run(user_ids, item_ids, P, Q)


def kernel(user_ids, item_ids, P, Q):
    out = _gmf(user_ids, item_ids, P, Q)
    return out.reshape(BATCH, 1)
